# Initial kernel scaffold; baseline (speedup 1.0000x reference)
import jax
import jax.numpy as jnp
from jax.experimental import pallas as pl


def kernel(source, query, fun, W0, b0, W1, b1, W2, b2):
    raise NotImplementedError("write your pallas kernel here")


# speedup vs baseline: 2.0419x; 2.0419x over previous
import functools

import jax
import jax.numpy as jnp
from jax.experimental import pallas as pl
from jax.experimental.pallas import tpu as pltpu

_RADIUS = 0.1
_QB = 8
_SB = 1000


def _dense_body(nsb, x_ref, y_ref, fun_ref, w0y_ref, w0x_ref, w0f_ref,
                b0_ref, w1_ref, b1_ref, w2_ref, b2_ref, out_ref,
                num_acc, deg_acc):
    sb = pl.program_id(1)
    qb_n = x_ref.shape[0]
    q_n = y_ref.shape[0]
    x = x_ref[...]
    y = y_ref[...]
    fun = fun_ref[...]
    f32 = jnp.float32

    dot = functools.partial(jax.lax.dot_general,
                            dimension_numbers=(((1,), (0,)), ((), ())),
                            preferred_element_type=f32,
                            precision=jax.lax.Precision.HIGHEST)

    a_blk = dot(x, w0x_ref[...]) + dot(fun, w0f_ref[...]) + b0_ref[...]
    c_blk = dot(y, w0y_ref[...])

    two_h = a_blk.shape[1]
    p_n = qb_n * q_n
    ap = jnp.broadcast_to(a_blk[:, None, :], (qb_n, q_n, two_h))
    ap = ap.reshape(p_n, two_h)
    cp = jnp.broadcast_to(c_blk[None, :, :], (qb_n, q_n, two_h))
    cp = cp.reshape(p_n, two_h)
    h = jax.nn.gelu(ap + cp)
    h = jax.nn.gelu(dot(h, w1_ref[...]) + b1_ref[...])
    h = dot(h, w2_ref[...]) + b2_ref[...]
    f_ch = h.shape[1]
    fp = jnp.broadcast_to(fun[:, None, :], (qb_n, q_n, f_ch)).reshape(p_n, f_ch)
    h = h * fp

    nx = jnp.sum(x * x, axis=1, keepdims=True)
    ny = jnp.sum(y * y, axis=1, keepdims=True)
    one_s = jnp.ones_like(nx)
    one_q = jnp.ones_like(ny)
    xc = jnp.concatenate([x, nx, one_s], axis=1)
    yc = jnp.concatenate([-2.0 * y, one_q, ny], axis=1)
    xp = jnp.broadcast_to(xc[:, None, :], (qb_n, q_n, 5)).reshape(p_n, 5)
    yp = jnp.broadcast_to(yc[None, :, :], (qb_n, q_n, 5)).reshape(p_n, 5)
    d2 = jnp.sum(xp * yp, axis=1, keepdims=True)
    maskf = (d2 <= _RADIUS * _RADIUS).astype(f32)

    h = h * maskf
    num_step = h.reshape(qb_n, q_n, f_ch).sum(axis=0)
    deg_step = maskf.reshape(qb_n, q_n, 1).sum(axis=0)
    deg_step = jnp.broadcast_to(deg_step, (q_n, f_ch))

    @pl.when(sb == 0)
    def _init():
        num_acc[...] = num_step
        deg_acc[...] = deg_step

    @pl.when(sb > 0)
    def _acc():
        num_acc[...] += num_step
        deg_acc[...] += deg_step

    @pl.when(sb == nsb - 1)
    def _final():
        out_ref[...] = num_acc[...] / jnp.maximum(deg_acc[...], 1.0)


def kernel(source, query, fun, W0, b0, W1, b1, W2, b2):
    x = source[0]
    y = query[0]
    n_src = x.shape[0]
    n_qry = y.shape[0]
    f_ch = fun.shape[1]
    w0y, w0x, w0f = W0[0:3], W0[3:6], W0[6:]

    nqb = n_qry // _QB
    nsb = n_src // _SB
    grid = (nqb, nsb)

    full = lambda a: pl.BlockSpec(a.shape, lambda qb, sb: (0,) * a.ndim)
    out = pl.pallas_call(
        functools.partial(_dense_body, nsb),
        grid=grid,
        in_specs=[
            pl.BlockSpec((_SB, x.shape[1]), lambda qb, sb: (sb, 0)),
            pl.BlockSpec((_QB, y.shape[1]), lambda qb, sb: (qb, 0)),
            pl.BlockSpec((_SB, f_ch), lambda qb, sb: (sb, 0)),
            full(w0y), full(w0x), full(w0f), full(b0),
            full(W1), full(b1), full(W2), full(b2),
        ],
        out_specs=pl.BlockSpec((_QB, f_ch), lambda qb, sb: (qb, 0)),
        out_shape=jax.ShapeDtypeStruct((n_qry, f_ch), jnp.float32),
        scratch_shapes=[
            pltpu.VMEM((_QB, f_ch), jnp.float32),
            pltpu.VMEM((_QB, f_ch), jnp.float32),
        ],
    )(x, y, fun, w0y, w0x, w0f, b0, W1, b1, W2, b2)
    return out


# speedup vs baseline: 16.7722x; 8.2138x over previous
import functools

import jax
import jax.numpy as jnp
from jax import lax
from jax.experimental import pallas as pl
from jax.experimental.pallas import tpu as pltpu
from jax.experimental.pallas import tpu_sc as plsc

_R2 = 0.01
_N = 10000
_NP = 10240
_NW = 32
_RPW = _NP // _NW
_CAP = 14336
_EB = 128
_NBLK = _CAP // _EB
_HIGH = jax.lax.Precision.HIGHEST


def _dot(a, b):
    return jax.lax.dot_general(a, b, (((1,), (0,)), ((), ())),
                               preferred_element_type=jnp.float32,
                               precision=_HIGH)


def _k1_body(x_ref, f_ref, y_ref, w0x_ref, w0f_ref, b0_ref, w0y_ref,
             a_ref, c_ref):
    a_ref[...] = (_dot(x_ref[...], w0x_ref[...]) +
                  _dot(f_ref[...], w0f_ref[...]) + b0_ref[...])
    c_ref[...] = _dot(y_ref[...], w0y_ref[...])


def _k2_body(nsb, y_ref, xt_ref, mask_ref, deg_ref, deg_acc):
    sb = pl.program_id(1)
    y = y_ref[...]
    d2 = jnp.zeros((y.shape[0], xt_ref.shape[1]), jnp.float32)
    for c in range(3):
        dx = y[:, c:c + 1] - xt_ref[c:c + 1, :]
        d2 = d2 + dx * dx
    m = (d2 <= _R2)
    mask_ref[...] = m.astype(jnp.int8)
    degs = jnp.sum(m.astype(jnp.float32), axis=1, keepdims=True)
    degs = jnp.broadcast_to(degs, (y.shape[0], 128))

    @pl.when(sb == 0)
    def _i():
        deg_acc[...] = degs

    @pl.when(sb > 0)
    def _a():
        deg_acc[...] += degs

    @pl.when(sb == nsb - 1)
    def _f():
        deg_ref[...] = deg_acc[...]


def _k3_body(mask_h, a_h, c_h, f_h, hbuf_h, fbuf_h, edst_h,
             row_v, esrc_v, edst_v, arows, crows, frows, sem):
    wid = lax.axis_index("s") * 2 + lax.axis_index("c")
    row0 = wid * _RPW
    lanes = lax.broadcasted_iota(jnp.int32, (16,), 0)
    zero16 = jnp.zeros((16,), jnp.int32)

    def fill(i, _):
        esrc_v[pl.ds(i * 16, 16)] = zero16 + _N
        edst_v[pl.ds(i * 16, 16)] = zero16 + row0
        return 0
    lax.fori_loop(0, _CAP // 16, fill, 0, unroll=4)

    def row_body(r, off):
        j = row0 + r
        pltpu.sync_copy(mask_h.at[pl.ds(j * (_NP // 4), _NP // 4)], row_v)

        def grp(g, off):
            wv = row_v[pl.ds(g * 16, 16)]
            anyhit = plsc.all_reduce_population_count(wv != 0)[0]

            def do(o):
                for k in range(4):
                    byte = (wv >> (8 * k)) & 0xFF
                    hit = byte != 0
                    idx = g * 64 + lanes * 4 + k
                    plsc.store_compressed(esrc_v.at[pl.ds(o, 16)], idx, mask=hit)
                    plsc.store_compressed(edst_v.at[pl.ds(o, 16)],
                                          zero16 + j, mask=hit)
                    pc = plsc.all_reduce_population_count(hit)[0]
                    o = jnp.minimum(o + pc, _CAP - 16)
                return o

            return lax.cond(anyhit > 0, do, lambda o: o, off)

        return lax.fori_loop(0, _NP // 64, grp, off)

    lax.fori_loop(0, _RPW, row_body, jnp.int32(0))
    pltpu.sync_copy(edst_v, edst_h.at[pl.ds(wid * _CAP, _CAP)])

    def gblk(b, _):
        idxs = esrc_v.at[pl.ds(b * _EB, _EB)]
        dsts = edst_v.at[pl.ds(b * _EB, _EB)]
        cp1 = pltpu.async_copy(a_h.at[idxs], arows, sem)
        cp1.wait()
        cp2 = pltpu.async_copy(c_h.at[dsts], crows, sem)
        cp2.wait()
        cp3 = pltpu.async_copy(f_h.at[idxs], frows, sem)
        cp3.wait()

        def addv(e, _):
            for k in range(16):
                s = pl.ds(k * 16, 16)
                arows[e, s] = arows[e, s] + crows[e, s]
            return 0
        lax.fori_loop(0, _EB, addv, 0)

        base = wid * _CAP + b * _EB
        pltpu.sync_copy(arows, hbuf_h.at[pl.ds(base, _EB)])
        pltpu.sync_copy(frows, fbuf_h.at[pl.ds(base, _EB)])
        return 0
    lax.fori_loop(0, _NBLK, gblk, 0)


def _k4_body(h_ref, f_ref, dst_ref, deg_ref, w1_ref, b1_ref, w2_ref,
             b2_ref, out_ref, acc):
    w = pl.program_id(0)
    b = pl.program_id(1)
    h = jax.nn.gelu(h_ref[...])
    h = jax.nn.gelu(_dot(h, w1_ref[...]) + b1_ref[...])
    h2 = (_dot(h, w2_ref[...]) + b2_ref[...]) * f_ref[...]

    dstv = dst_ref[...].reshape(1, _EB) - w * _RPW
    rows = lax.broadcasted_iota(jnp.int32, (_RPW, 1), 0)
    onehot = (rows == dstv).astype(jnp.float32)
    contrib = _dot(onehot, h2)

    @pl.when(b == 0)
    def _i():
        acc[...] = contrib

    @pl.when(b > 0)
    def _a():
        acc[...] += contrib

    @pl.when(b == _NBLK - 1)
    def _f():
        out_ref[...] = acc[...] / jnp.maximum(deg_ref[...], 1.0)


def kernel(source, query, fun, W0, b0, W1, b1, W2, b2):
    x = source[0]
    y = query[0]
    f_ch = fun.shape[1]
    pad_n = _NP - _N
    xp = jnp.concatenate(
        [x, jnp.full((pad_n, 3), 999.0, jnp.float32)], axis=0)
    yp = jnp.concatenate(
        [y, jnp.full((pad_n, 3), 999.0, jnp.float32)], axis=0)
    funp = jnp.concatenate(
        [fun, jnp.zeros((pad_n, f_ch), jnp.float32)], axis=0)
    xt8 = jnp.concatenate([xp.T, jnp.zeros((5, _NP), jnp.float32)], axis=0)
    w0y, w0x, w0f = W0[0:3], W0[3:6], W0[6:]

    nb1 = _NP // 1024
    full = lambda a: pl.BlockSpec(a.shape, lambda *_: (0,) * a.ndim)
    a_mat, c_mat = pl.pallas_call(
        _k1_body,
        grid=(nb1,),
        in_specs=[
            pl.BlockSpec((1024, 3), lambda i: (i, 0)),
            pl.BlockSpec((1024, f_ch), lambda i: (i, 0)),
            pl.BlockSpec((1024, 3), lambda i: (i, 0)),
            full(w0x), full(w0f), full(b0), full(w0y),
        ],
        out_specs=[
            pl.BlockSpec((1024, 256), lambda i: (i, 0)),
            pl.BlockSpec((1024, 256), lambda i: (i, 0)),
        ],
        out_shape=[
            jax.ShapeDtypeStruct((_NP, 256), jnp.float32),
            jax.ShapeDtypeStruct((_NP, 256), jnp.float32),
        ],
    )(xp, funp, yp, w0x, w0f, b0, w0y)

    nsb = 5
    sbw = _NP // nsb
    mask, deg2d = pl.pallas_call(
        functools.partial(_k2_body, nsb),
        grid=(_NW, nsb),
        in_specs=[
            pl.BlockSpec((_RPW, 3), lambda qb, sb: (qb, 0)),
            pl.BlockSpec((8, sbw), lambda qb, sb: (0, sb)),
        ],
        out_specs=[
            pl.BlockSpec((_RPW, sbw), lambda qb, sb: (qb, sb)),
            pl.BlockSpec((_RPW, 128), lambda qb, sb: (qb, 0)),
        ],
        out_shape=[
            jax.ShapeDtypeStruct((_NP, _NP), jnp.int8),
            jax.ShapeDtypeStruct((_NP, 128), jnp.float32),
        ],
        scratch_shapes=[pltpu.VMEM((_RPW, 128), jnp.float32)],
    )(yp, xt8)

    mesh = plsc.VectorSubcoreMesh(core_axis_name="c", subcore_axis_name="s")
    k3 = pl.kernel(
        _k3_body,
        mesh=mesh,
        out_type=[
            jax.ShapeDtypeStruct((_NW * _CAP, 256), jnp.float32),
            jax.ShapeDtypeStruct((_NW * _CAP, f_ch), jnp.float32),
            jax.ShapeDtypeStruct((_NW * _CAP,), jnp.int32),
        ],
        scratch_types=[
            pltpu.VMEM((_NP // 4,), jnp.int32),
            pltpu.VMEM((_CAP,), jnp.int32),
            pltpu.VMEM((_CAP,), jnp.int32),
            pltpu.VMEM((_EB, 256), jnp.float32),
            pltpu.VMEM((_EB, 256), jnp.float32),
            pltpu.VMEM((_EB, f_ch), jnp.float32),
            pltpu.SemaphoreType.DMA,
        ],
        compiler_params=pltpu.CompilerParams(needs_layout_passes=False),
    )
    mask_w = jax.lax.bitcast_convert_type(
        mask.reshape(_NP * _NP // 4, 4), jnp.int32)
    hbuf, fbuf, edst = k3(mask_w, a_mat, c_mat, funp)
    edst3 = edst.reshape(_NW * _NBLK, 1, _EB)

    outp = pl.pallas_call(
        _k4_body,
        grid=(_NW, _NBLK),
        in_specs=[
            pl.BlockSpec((_EB, 256), lambda w, b: (w * _NBLK + b, 0)),
            pl.BlockSpec((_EB, f_ch), lambda w, b: (w * _NBLK + b, 0)),
            pl.BlockSpec((1, 1, _EB), lambda w, b: (w * _NBLK + b, 0, 0)),
            pl.BlockSpec((_RPW, 128), lambda w, b: (w, 0)),
            full(W1), full(b1), full(W2), full(b2),
        ],
        out_specs=pl.BlockSpec((_RPW, f_ch), lambda w, b: (w, 0)),
        out_shape=jax.ShapeDtypeStruct((_NP, f_ch), jnp.float32),
        scratch_shapes=[pltpu.VMEM((_RPW, f_ch), jnp.float32)],
    )(hbuf, fbuf, edst3, deg2d, W1, b1, W2, b2)
    return outp[:_N]


# speedup vs baseline: 60.2464x; 3.5920x over previous
import functools

import jax
import jax.numpy as jnp
from jax import lax
from jax.experimental import pallas as pl
from jax.experimental.pallas import tpu as pltpu
from jax.experimental.pallas import tpu_sc as plsc

_R2 = 0.01
_N = 10000
_NP = 10240
_NW = 32
_RPW = _NP // _NW
_CAP = 14336
_EB = 128
_NBLK = _CAP // _EB
_EB4 = 512
_NBLK4 = _CAP // _EB4
_HIGH = jax.lax.Precision.HIGHEST


def _dot(a, b):
    return jax.lax.dot_general(a, b, (((1,), (0,)), ((), ())),
                               preferred_element_type=jnp.float32,
                               precision=_HIGH)


def _k1_body(x_ref, f_ref, y_ref, w0x_ref, w0f_ref, b0_ref, w0y_ref,
             a_ref, c_ref):
    a_ref[...] = (_dot(x_ref[...], w0x_ref[...]) +
                  _dot(f_ref[...], w0f_ref[...]) + b0_ref[...])
    c_ref[...] = _dot(y_ref[...], w0y_ref[...])


def _k2_body(nsb, y_ref, xt_ref, mask_ref, deg_ref, deg_acc):
    sb = pl.program_id(1)
    y = y_ref[...]
    d2 = jnp.zeros((y.shape[0], xt_ref.shape[1]), jnp.float32)
    for c in range(3):
        dx = y[:, c:c + 1] - xt_ref[c:c + 1, :]
        d2 = d2 + dx * dx
    m = (d2 <= _R2)
    mask_ref[...] = m.astype(jnp.int32)
    degs = jnp.sum(m.astype(jnp.float32), axis=1, keepdims=True)
    degs = jnp.broadcast_to(degs, (y.shape[0], 128))

    @pl.when(sb == 0)
    def _i():
        deg_acc[...] = degs

    @pl.when(sb > 0)
    def _a():
        deg_acc[...] += degs

    @pl.when(sb == nsb - 1)
    def _f():
        deg_ref[...] = deg_acc[...]


def _k3_body(mask_h, a_h, f_h, hbuf_h, fbuf_h, edst_h,
             tbuf, esrc_v, edst_v, arows, frows, sem, sem2):
    wid = lax.axis_index("s") * 2 + lax.axis_index("c")
    row0 = wid * _RPW
    lanes = lax.broadcasted_iota(jnp.int32, (16,), 0)
    zero16 = jnp.zeros((16,), jnp.int32)

    def fill(i, _):
        esrc_v[pl.ds(i * 16, 16)] = zero16 + _N
        edst_v[pl.ds(i * 16, 16)] = zero16 + row0
        return 0
    lax.fori_loop(0, _CAP // 16, fill, 0, unroll=4)

    def rowtile(rt, off):
        def colblk(cb, off):
            pltpu.sync_copy(
                mask_h.at[pl.ds(row0 + rt * 8, 8), pl.ds(cb * 1280, 1280)],
                tbuf)

            def rowin(r8, off):
                j = row0 + rt * 8 + r8

                def grp(g, off):
                    w0 = tbuf[r8, pl.ds(g * 64, 16)]
                    w1 = tbuf[r8, pl.ds(g * 64 + 16, 16)]
                    w2 = tbuf[r8, pl.ds(g * 64 + 32, 16)]
                    w3 = tbuf[r8, pl.ds(g * 64 + 48, 16)]
                    comb = w0 | w1 | w2 | w3
                    anyhit = plsc.all_reduce_population_count(comb != 0)[0]

                    def do(o):
                        for k, wk in enumerate((w0, w1, w2, w3)):
                            hit = wk != 0
                            idx = cb * 1280 + g * 64 + k * 16 + lanes
                            plsc.store_compressed(
                                esrc_v.at[pl.ds(o, 16)], idx, mask=hit)
                            plsc.store_compressed(
                                edst_v.at[pl.ds(o, 16)], zero16 + j, mask=hit)
                            pc = plsc.all_reduce_population_count(hit)[0]
                            o = jnp.minimum(o + pc, _CAP - 16)
                        return o

                    return lax.cond(anyhit > 0, do, lambda o: o, off)

                return lax.fori_loop(0, 20, grp, off)

            return lax.fori_loop(0, 8, rowin, off)

        return lax.fori_loop(0, 8, colblk, off)

    lax.fori_loop(0, _RPW // 8, rowtile, jnp.int32(0))
    pltpu.sync_copy(edst_v, edst_h.at[pl.ds(wid * _CAP, _CAP)])

    def gblk(b, _):
        idxs = esrc_v.at[pl.ds(b * _EB, _EB)]
        cp1 = pltpu.async_copy(a_h.at[idxs], arows, sem)
        cp2 = pltpu.async_copy(f_h.at[idxs], frows, sem2)
        cp1.wait()
        cp2.wait()
        base = wid * _CAP + b * _EB
        pltpu.sync_copy(arows, hbuf_h.at[pl.ds(base, _EB)])
        pltpu.sync_copy(frows, fbuf_h.at[pl.ds(base, _EB)])
        return 0
    lax.fori_loop(0, _NBLK, gblk, 0)


def _k4_body(h_ref, f_ref, dst_ref, c_ref, deg_ref, w1_ref, b1_ref,
             w2_ref, b2_ref, out_ref, acc):
    b = pl.program_id(1)
    w = pl.program_id(0)
    dstv = dst_ref[...].reshape(1, _EB4) - w * _RPW
    rows = lax.broadcasted_iota(jnp.int32, (_RPW, 1), 0)
    onehot = (rows == dstv).astype(jnp.float32)

    cexp = jax.lax.dot_general(onehot, c_ref[...], (((0,), (0,)), ((), ())),
                               preferred_element_type=jnp.float32,
                               precision=_HIGH)
    h = jax.nn.gelu(h_ref[...] + cexp)
    h = jax.nn.gelu(_dot(h, w1_ref[...]) + b1_ref[...])
    h2 = (_dot(h, w2_ref[...]) + b2_ref[...]) * f_ref[...]
    contrib = _dot(onehot, h2)

    @pl.when(b == 0)
    def _i():
        acc[...] = contrib

    @pl.when(b > 0)
    def _a():
        acc[...] += contrib

    @pl.when(b == _NBLK4 - 1)
    def _f():
        out_ref[...] = acc[...] / jnp.maximum(deg_ref[...], 1.0)


def kernel(source, query, fun, W0, b0, W1, b1, W2, b2):
    x = source[0]
    y = query[0]
    f_ch = fun.shape[1]
    pad_n = _NP - _N
    xp = jnp.concatenate(
        [x, jnp.full((pad_n, 3), 999.0, jnp.float32)], axis=0)
    yp = jnp.concatenate(
        [y, jnp.full((pad_n, 3), 999.0, jnp.float32)], axis=0)
    funp = jnp.concatenate(
        [fun, jnp.zeros((pad_n, f_ch), jnp.float32)], axis=0)
    xt8 = jnp.concatenate([xp.T, jnp.zeros((5, _NP), jnp.float32)], axis=0)
    w0y, w0x, w0f = W0[0:3], W0[3:6], W0[6:]

    nb1 = _NP // 1024
    full = lambda a: pl.BlockSpec(a.shape, lambda *_: (0,) * a.ndim)
    a_mat, c_mat = pl.pallas_call(
        _k1_body,
        grid=(nb1,),
        in_specs=[
            pl.BlockSpec((1024, 3), lambda i: (i, 0)),
            pl.BlockSpec((1024, f_ch), lambda i: (i, 0)),
            pl.BlockSpec((1024, 3), lambda i: (i, 0)),
            full(w0x), full(w0f), full(b0), full(w0y),
        ],
        out_specs=[
            pl.BlockSpec((1024, 256), lambda i: (i, 0)),
            pl.BlockSpec((1024, 256), lambda i: (i, 0)),
        ],
        out_shape=[
            jax.ShapeDtypeStruct((_NP, 256), jnp.float32),
            jax.ShapeDtypeStruct((_NP, 256), jnp.float32),
        ],
    )(xp, funp, yp, w0x, w0f, b0, w0y)

    nsb = 5
    sbw = _NP // nsb
    mask, deg2d = pl.pallas_call(
        functools.partial(_k2_body, nsb),
        grid=(_NW, nsb),
        in_specs=[
            pl.BlockSpec((_RPW, 3), lambda qb, sb: (qb, 0)),
            pl.BlockSpec((8, sbw), lambda qb, sb: (0, sb)),
        ],
        out_specs=[
            pl.BlockSpec((_RPW, sbw), lambda qb, sb: (qb, sb)),
            pl.BlockSpec((_RPW, 128), lambda qb, sb: (qb, 0)),
        ],
        out_shape=[
            jax.ShapeDtypeStruct((_NP, _NP), jnp.int32),
            jax.ShapeDtypeStruct((_NP, 128), jnp.float32),
        ],
        scratch_shapes=[pltpu.VMEM((_RPW, 128), jnp.float32)],
    )(yp, xt8)

    mesh = plsc.VectorSubcoreMesh(core_axis_name="c", subcore_axis_name="s")
    k3 = pl.kernel(
        _k3_body,
        mesh=mesh,
        out_type=[
            jax.ShapeDtypeStruct((_NW * _CAP, 256), jnp.float32),
            jax.ShapeDtypeStruct((_NW * _CAP, f_ch), jnp.float32),
            jax.ShapeDtypeStruct((_NW * _CAP,), jnp.int32),
        ],
        scratch_types=[
            pltpu.VMEM((8, 1280), jnp.int32),
            pltpu.VMEM((_CAP,), jnp.int32),
            pltpu.VMEM((_CAP,), jnp.int32),
            pltpu.VMEM((_EB, 256), jnp.float32),
            pltpu.VMEM((_EB, f_ch), jnp.float32),
            pltpu.SemaphoreType.DMA,
            pltpu.SemaphoreType.DMA,
        ],
        compiler_params=pltpu.CompilerParams(needs_layout_passes=False),
    )
    hbuf, fbuf, edst = k3(mask, a_mat, funp)
    edst3 = edst.reshape(_NW * _NBLK4, 1, _EB4)

    outp = pl.pallas_call(
        _k4_body,
        grid=(_NW, _NBLK4),
        in_specs=[
            pl.BlockSpec((_EB4, 256), lambda w, b: (w * _NBLK4 + b, 0)),
            pl.BlockSpec((_EB4, f_ch), lambda w, b: (w * _NBLK4 + b, 0)),
            pl.BlockSpec((1, 1, _EB4), lambda w, b: (w * _NBLK4 + b, 0, 0)),
            pl.BlockSpec((_RPW, 256), lambda w, b: (w, 0)),
            pl.BlockSpec((_RPW, 128), lambda w, b: (w, 0)),
            full(W1), full(b1), full(W2), full(b2),
        ],
        out_specs=pl.BlockSpec((_RPW, f_ch), lambda w, b: (w, 0)),
        out_shape=jax.ShapeDtypeStruct((_NP, f_ch), jnp.float32),
        scratch_shapes=[pltpu.VMEM((_RPW, f_ch), jnp.float32)],
    )(hbuf, fbuf, edst3, c_mat, deg2d, W1, b1, W2, b2)
    return outp[:_N]


# speedup vs baseline: 73.7117x; 1.2235x over previous
import functools

import jax
import jax.numpy as jnp
from jax import lax
from jax.experimental import pallas as pl
from jax.experimental.pallas import tpu as pltpu
from jax.experimental.pallas import tpu_sc as plsc

_R2 = 0.01
_N = 10000
_NP = 10240
_NW = 32
_RPW = _NP // _NW
_CAP = 14336
_EB = 64
_NBLK = _CAP // _EB
_EB4 = 512
_NBLK4 = _CAP // _EB4
_HIGH = jax.lax.Precision.HIGHEST


def _dot(a, b):
    return jax.lax.dot_general(a, b, (((1,), (0,)), ((), ())),
                               preferred_element_type=jnp.float32,
                               precision=_HIGH)


def _k1_body(x_ref, f_ref, y_ref, w0x_ref, w0f_ref, b0_ref, w0y_ref,
             a_ref, c_ref):
    a_ref[...] = (_dot(x_ref[...], w0x_ref[...]) +
                  _dot(f_ref[...], w0f_ref[...]) + b0_ref[...])
    c_ref[...] = _dot(y_ref[...], w0y_ref[...])


def _k2_body(nsb, y_ref, xt_ref, mask_ref, deg_ref, deg_acc):
    sb = pl.program_id(1)
    y = y_ref[...]
    d2 = jnp.zeros((y.shape[0], xt_ref.shape[1]), jnp.float32)
    for c in range(3):
        dx = y[:, c:c + 1] - xt_ref[c:c + 1, :]
        d2 = d2 + dx * dx
    m = (d2 <= _R2)
    mask_ref[...] = m.astype(jnp.int32)
    degs = jnp.sum(m.astype(jnp.float32), axis=1, keepdims=True)
    degs = jnp.broadcast_to(degs, (y.shape[0], 128))

    @pl.when(sb == 0)
    def _i():
        deg_acc[...] = degs

    @pl.when(sb > 0)
    def _a():
        deg_acc[...] += degs

    @pl.when(sb == nsb - 1)
    def _f():
        deg_ref[...] = deg_acc[...]


def _k3_body(mask_h, a_h, f_h, hbuf_h, fbuf_h, edst_h,
             tbuf, esrc_v, edst_v, arows, frows, arows2, frows2,
             sem, sem2, sem3, sem4):
    wid = lax.axis_index("s") * 2 + lax.axis_index("c")
    row0 = wid * _RPW
    lanes = lax.broadcasted_iota(jnp.int32, (16,), 0)
    zero16 = jnp.zeros((16,), jnp.int32)

    def fill(i, _):
        esrc_v[pl.ds(i * 16, 16)] = zero16 + _N
        edst_v[pl.ds(i * 16, 16)] = zero16 + row0
        return 0
    lax.fori_loop(0, _CAP // 16, fill, 0, unroll=4)

    def rowtile(rt, off):
        def colblk(cb, off):
            pltpu.sync_copy(
                mask_h.at[pl.ds(row0 + rt * 8, 8), pl.ds(cb * 1280, 1280)],
                tbuf)

            def rowin(r8, off):
                j = row0 + rt * 8 + r8

                def grp(g, off):
                    w0 = tbuf[r8, pl.ds(g * 64, 16)]
                    w1 = tbuf[r8, pl.ds(g * 64 + 16, 16)]
                    w2 = tbuf[r8, pl.ds(g * 64 + 32, 16)]
                    w3 = tbuf[r8, pl.ds(g * 64 + 48, 16)]
                    comb = w0 | w1 | w2 | w3
                    anyhit = plsc.all_reduce_population_count(comb != 0)[0]

                    def do(o):
                        for k, wk in enumerate((w0, w1, w2, w3)):
                            hit = wk != 0
                            idx = cb * 1280 + g * 64 + k * 16 + lanes
                            plsc.store_compressed(
                                esrc_v.at[pl.ds(o, 16)], idx, mask=hit)
                            plsc.store_compressed(
                                edst_v.at[pl.ds(o, 16)], zero16 + j, mask=hit)
                            pc = plsc.all_reduce_population_count(hit)[0]
                            o = jnp.minimum(o + pc, _CAP - 16)
                        return o

                    return lax.cond(anyhit > 0, do, lambda o: o, off)

                return lax.fori_loop(0, 20, grp, off)

            return lax.fori_loop(0, 8, rowin, off)

        return lax.fori_loop(0, 8, colblk, off)

    lax.fori_loop(0, _RPW // 8, rowtile, jnp.int32(0))
    pltpu.sync_copy(edst_v, edst_h.at[pl.ds(wid * _CAP, _CAP)])

    def fire(b, ar, fr, sa, sf):
        idxs = esrc_v.at[pl.ds(b * _EB, _EB)]
        pltpu.async_copy(a_h.at[idxs], ar, sa)
        pltpu.async_copy(f_h.at[idxs], fr, sf)

    def drain(b, ar, fr, sa, sf):
        pltpu.make_async_copy(a_h.at[pl.ds(0, _EB)], ar, sa).wait()
        pltpu.make_async_copy(f_h.at[pl.ds(0, _EB)], fr, sf).wait()
        base = wid * _CAP + b * _EB
        pltpu.sync_copy(ar, hbuf_h.at[pl.ds(base, _EB)])
        pltpu.sync_copy(fr, fbuf_h.at[pl.ds(base, _EB)])

    fire(0, arows, frows, sem, sem2)

    def gpair(t2, _):
        b = t2 * 2
        fire(b + 1, arows2, frows2, sem3, sem4)
        drain(b, arows, frows, sem, sem2)
        fire(b + 2, arows, frows, sem, sem2)
        drain(b + 1, arows2, frows2, sem3, sem4)
        return 0
    lax.fori_loop(0, _NBLK // 2 - 1, gpair, 0)
    b = _NBLK - 2
    fire(b + 1, arows2, frows2, sem3, sem4)
    drain(b, arows, frows, sem, sem2)
    drain(b + 1, arows2, frows2, sem3, sem4)


def _k4_body(h_ref, f_ref, dst_ref, c_ref, deg_ref, w1_ref, b1_ref,
             w2_ref, b2_ref, out_ref, acc):
    b = pl.program_id(1)
    w = pl.program_id(0)
    f32 = jnp.float32
    dstv = dst_ref[...].reshape(1, _EB4) - w * _RPW
    rows = lax.broadcasted_iota(jnp.int32, (_RPW, 1), 0)
    onehot = (rows == dstv).astype(f32)

    cexp = jax.lax.dot_general(onehot, c_ref[...], (((0,), (0,)), ((), ())),
                               preferred_element_type=f32)
    dot_h = functools.partial(jax.lax.dot_general,
                              dimension_numbers=(((1,), (0,)), ((), ())),
                              preferred_element_type=f32,
                              precision=_HIGH)
    h = jax.nn.gelu(h_ref[...] + cexp)
    h = jax.nn.gelu(dot_h(h, w1_ref[...]) + b1_ref[...])
    h2 = (dot_h(h, w2_ref[...]) + b2_ref[...]) * f_ref[...]
    contrib = jax.lax.dot_general(onehot, h2, (((1,), (0,)), ((), ())),
                                  preferred_element_type=f32)

    @pl.when(b == 0)
    def _i():
        acc[...] = contrib

    @pl.when(b > 0)
    def _a():
        acc[...] += contrib

    @pl.when(b == _NBLK4 - 1)
    def _f():
        out_ref[...] = acc[...] / jnp.maximum(deg_ref[...], 1.0)


def kernel(source, query, fun, W0, b0, W1, b1, W2, b2):
    x = source[0]
    y = query[0]
    f_ch = fun.shape[1]
    pad_n = _NP - _N
    xp = jnp.concatenate(
        [x, jnp.full((pad_n, 3), 999.0, jnp.float32)], axis=0)
    yp = jnp.concatenate(
        [y, jnp.full((pad_n, 3), 999.0, jnp.float32)], axis=0)
    funp = jnp.concatenate(
        [fun, jnp.zeros((pad_n, f_ch), jnp.float32)], axis=0)
    xt8 = jnp.concatenate([xp.T, jnp.zeros((5, _NP), jnp.float32)], axis=0)
    w0y, w0x, w0f = W0[0:3], W0[3:6], W0[6:]

    nb1 = _NP // 1024
    full = lambda a: pl.BlockSpec(a.shape, lambda *_: (0,) * a.ndim)
    a_mat, c_mat = pl.pallas_call(
        _k1_body,
        grid=(nb1,),
        in_specs=[
            pl.BlockSpec((1024, 3), lambda i: (i, 0)),
            pl.BlockSpec((1024, f_ch), lambda i: (i, 0)),
            pl.BlockSpec((1024, 3), lambda i: (i, 0)),
            full(w0x), full(w0f), full(b0), full(w0y),
        ],
        out_specs=[
            pl.BlockSpec((1024, 256), lambda i: (i, 0)),
            pl.BlockSpec((1024, 256), lambda i: (i, 0)),
        ],
        out_shape=[
            jax.ShapeDtypeStruct((_NP, 256), jnp.float32),
            jax.ShapeDtypeStruct((_NP, 256), jnp.float32),
        ],
    )(xp, funp, yp, w0x, w0f, b0, w0y)

    nsb = 5
    sbw = _NP // nsb
    mask, deg2d = pl.pallas_call(
        functools.partial(_k2_body, nsb),
        grid=(_NW, nsb),
        in_specs=[
            pl.BlockSpec((_RPW, 3), lambda qb, sb: (qb, 0)),
            pl.BlockSpec((8, sbw), lambda qb, sb: (0, sb)),
        ],
        out_specs=[
            pl.BlockSpec((_RPW, sbw), lambda qb, sb: (qb, sb)),
            pl.BlockSpec((_RPW, 128), lambda qb, sb: (qb, 0)),
        ],
        out_shape=[
            jax.ShapeDtypeStruct((_NP, _NP), jnp.int32),
            jax.ShapeDtypeStruct((_NP, 128), jnp.float32),
        ],
        scratch_shapes=[pltpu.VMEM((_RPW, 128), jnp.float32)],
    )(yp, xt8)

    mesh = plsc.VectorSubcoreMesh(core_axis_name="c", subcore_axis_name="s")
    k3 = pl.kernel(
        _k3_body,
        mesh=mesh,
        out_type=[
            jax.ShapeDtypeStruct((_NW * _CAP, 256), jnp.float32),
            jax.ShapeDtypeStruct((_NW * _CAP, f_ch), jnp.float32),
            jax.ShapeDtypeStruct((_NW * _CAP,), jnp.int32),
        ],
        scratch_types=[
            pltpu.VMEM((8, 1280), jnp.int32),
            pltpu.VMEM((_CAP,), jnp.int32),
            pltpu.VMEM((_CAP,), jnp.int32),
            pltpu.VMEM((_EB, 256), jnp.float32),
            pltpu.VMEM((_EB, f_ch), jnp.float32),
            pltpu.VMEM((_EB, 256), jnp.float32),
            pltpu.VMEM((_EB, f_ch), jnp.float32),
            pltpu.SemaphoreType.DMA,
            pltpu.SemaphoreType.DMA,
            pltpu.SemaphoreType.DMA,
            pltpu.SemaphoreType.DMA,
        ],
        compiler_params=pltpu.CompilerParams(needs_layout_passes=False),
    )
    hbuf, fbuf, edst = k3(mask, a_mat, funp)
    edst3 = edst.reshape(_NW * _NBLK4, 1, _EB4)

    outp = pl.pallas_call(
        _k4_body,
        grid=(_NW, _NBLK4),
        in_specs=[
            pl.BlockSpec((_EB4, 256), lambda w, b: (w * _NBLK4 + b, 0)),
            pl.BlockSpec((_EB4, f_ch), lambda w, b: (w * _NBLK4 + b, 0)),
            pl.BlockSpec((1, 1, _EB4), lambda w, b: (w * _NBLK4 + b, 0, 0)),
            pl.BlockSpec((_RPW, 256), lambda w, b: (w, 0)),
            pl.BlockSpec((_RPW, 128), lambda w, b: (w, 0)),
            full(W1), full(b1), full(W2), full(b2),
        ],
        out_specs=pl.BlockSpec((_RPW, f_ch), lambda w, b: (w, 0)),
        out_shape=jax.ShapeDtypeStruct((_NP, f_ch), jnp.float32),
        scratch_shapes=[pltpu.VMEM((_RPW, f_ch), jnp.float32)],
    )(hbuf, fbuf, edst3, c_mat, deg2d, W1, b1, W2, b2)
    return outp[:_N]


# speedup vs baseline: 76.7907x; 1.0418x over previous
import functools

import jax
import jax.numpy as jnp
from jax import lax
from jax.experimental import pallas as pl
from jax.experimental.pallas import tpu as pltpu
from jax.experimental.pallas import tpu_sc as plsc

_R2 = 0.01
_N = 10000
_NP = 10240
_NW = 32
_RPW = _NP // _NW
_CAP = 14336
_EB = 64
_NBLK = _CAP // _EB
_EB4 = 512
_NBLK4 = _CAP // _EB4
_HIGH = jax.lax.Precision.HIGHEST


def _dot(a, b):
    return jax.lax.dot_general(a, b, (((1,), (0,)), ((), ())),
                               preferred_element_type=jnp.float32,
                               precision=_HIGH)


def _k1_body(x_ref, f_ref, y_ref, w0x_ref, w0f_ref, b0_ref, w0y_ref,
             a_ref, c_ref):
    a_ref[...] = (_dot(x_ref[...], w0x_ref[...]) +
                  _dot(f_ref[...], w0f_ref[...]) + b0_ref[...])
    c_ref[...] = _dot(y_ref[...], w0y_ref[...])


def _k2_body(nsb, y_ref, xt_ref, mask_ref, deg_ref, deg_acc):
    sb = pl.program_id(1)
    y = y_ref[...]
    d2 = jnp.zeros((y.shape[0], xt_ref.shape[1]), jnp.float32)
    for c in range(3):
        dx = y[:, c:c + 1] - xt_ref[c:c + 1, :]
        d2 = d2 + dx * dx
    m = (d2 <= _R2)
    mask_ref[...] = m.astype(jnp.int32)
    degs = jnp.sum(m.astype(jnp.float32), axis=1, keepdims=True)
    degs = jnp.broadcast_to(degs, (y.shape[0], 128))

    @pl.when(sb == 0)
    def _i():
        deg_acc[...] = degs

    @pl.when(sb > 0)
    def _a():
        deg_acc[...] += degs

    @pl.when(sb == nsb - 1)
    def _f():
        deg_ref[...] = deg_acc[...]


def _k3_body(mask_h, a_h, f_h, hbuf_h, fbuf_h, edst_h,
             tbuf, tbuf2, esrc_v, edst_v, arows, frows, arows2, frows2,
             sem, sem2, sem3, sem4, sem5, sem6):
    wid = lax.axis_index("s") * 2 + lax.axis_index("c")
    row0 = wid * _RPW
    lanes = lax.broadcasted_iota(jnp.int32, (16,), 0)
    zero16 = jnp.zeros((16,), jnp.int32)

    def fill(i, _):
        esrc_v[pl.ds(i * 16, 16)] = zero16 + _N
        edst_v[pl.ds(i * 16, 16)] = zero16 + row0
        return 0
    lax.fori_loop(0, _CAP // 16, fill, 0, unroll=4)

    n_tiles = (_RPW // 8) * 8

    def mfire(t, buf, s):
        rt = t // 8
        cb = t % 8
        pltpu.async_copy(
            mask_h.at[pl.ds(row0 + rt * 8, 8), pl.ds(cb * 1280, 1280)],
            buf, s)

    def mdrain(buf, s):
        pltpu.make_async_copy(
            mask_h.at[pl.ds(0, 8), pl.ds(0, 1280)], buf, s).wait()

    def process(t, buf, off):
        rt = t // 8
        cb = t % 8

        def rowin(r8, off):
            j = row0 + rt * 8 + r8

            def grp(g, off):
                w0 = buf[r8, pl.ds(g * 64, 16)]
                w1 = buf[r8, pl.ds(g * 64 + 16, 16)]
                w2 = buf[r8, pl.ds(g * 64 + 32, 16)]
                w3 = buf[r8, pl.ds(g * 64 + 48, 16)]
                comb = w0 | w1 | w2 | w3
                anyhit = plsc.all_reduce_population_count(comb != 0)[0]

                def do(o):
                    for k, wk in enumerate((w0, w1, w2, w3)):
                        hit = wk != 0
                        idx = cb * 1280 + g * 64 + k * 16 + lanes
                        plsc.store_compressed(
                            esrc_v.at[pl.ds(o, 16)], idx, mask=hit)
                        plsc.store_compressed(
                            edst_v.at[pl.ds(o, 16)], zero16 + j, mask=hit)
                        pc = plsc.all_reduce_population_count(hit)[0]
                        o = jnp.minimum(o + pc, _CAP - 16)
                    return o

                return lax.cond(anyhit > 0, do, lambda o: o, off)

            return lax.fori_loop(0, 20, grp, off)

        return lax.fori_loop(0, 8, rowin, off)

    mfire(0, tbuf, sem5)

    def mpair(t2, off):
        t = t2 * 2
        mfire(t + 1, tbuf2, sem6)
        mdrain(tbuf, sem5)
        off = process(t, tbuf, off)
        mfire(t + 2, tbuf, sem5)
        mdrain(tbuf2, sem6)
        return process(t + 1, tbuf2, off)

    off = lax.fori_loop(0, n_tiles // 2 - 1, mpair, jnp.int32(0))
    t = n_tiles - 2
    mfire(t + 1, tbuf2, sem6)
    mdrain(tbuf, sem5)
    off = process(t, tbuf, off)
    mdrain(tbuf2, sem6)
    off = process(t + 1, tbuf2, off)
    pltpu.sync_copy(edst_v, edst_h.at[pl.ds(wid * _CAP, _CAP)])

    def fire(b, ar, fr, sa, sf):
        idxs = esrc_v.at[pl.ds(b * _EB, _EB)]
        pltpu.async_copy(a_h.at[idxs], ar, sa)
        pltpu.async_copy(f_h.at[idxs], fr, sf)

    def drain(b, ar, fr, sa, sf):
        pltpu.make_async_copy(a_h.at[pl.ds(0, _EB)], ar, sa).wait()
        pltpu.make_async_copy(f_h.at[pl.ds(0, _EB)], fr, sf).wait()
        base = wid * _CAP + b * _EB
        pltpu.sync_copy(ar, hbuf_h.at[pl.ds(base, _EB)])
        pltpu.sync_copy(fr, fbuf_h.at[pl.ds(base, _EB)])

    fire(0, arows, frows, sem, sem2)

    def gpair(t2, _):
        b = t2 * 2
        fire(b + 1, arows2, frows2, sem3, sem4)
        drain(b, arows, frows, sem, sem2)
        fire(b + 2, arows, frows, sem, sem2)
        drain(b + 1, arows2, frows2, sem3, sem4)
        return 0
    lax.fori_loop(0, _NBLK // 2 - 1, gpair, 0)
    b = _NBLK - 2
    fire(b + 1, arows2, frows2, sem3, sem4)
    drain(b, arows, frows, sem, sem2)
    drain(b + 1, arows2, frows2, sem3, sem4)


def _k4_body(h_ref, f_ref, dst_ref, c_ref, deg_ref, w1_ref, b1_ref,
             w2_ref, b2_ref, out_ref, acc):
    b = pl.program_id(1)
    w = pl.program_id(0)
    f32 = jnp.float32
    dstv = dst_ref[...].reshape(1, _EB4) - w * _RPW
    rows = lax.broadcasted_iota(jnp.int32, (_RPW, 1), 0)
    onehot = (rows == dstv).astype(f32)

    cexp = jax.lax.dot_general(onehot, c_ref[...], (((0,), (0,)), ((), ())),
                               preferred_element_type=f32)
    dot_h = functools.partial(jax.lax.dot_general,
                              dimension_numbers=(((1,), (0,)), ((), ())),
                              preferred_element_type=f32,
                              precision=_HIGH)
    h = jax.nn.gelu(h_ref[...] + cexp)
    h = jax.nn.gelu(dot_h(h, w1_ref[...]) + b1_ref[...])
    h2 = (dot_h(h, w2_ref[...]) + b2_ref[...]) * f_ref[...]
    contrib = jax.lax.dot_general(onehot, h2, (((1,), (0,)), ((), ())),
                                  preferred_element_type=f32)

    @pl.when(b == 0)
    def _i():
        acc[...] = contrib

    @pl.when(b > 0)
    def _a():
        acc[...] += contrib

    @pl.when(b == _NBLK4 - 1)
    def _f():
        out_ref[...] = acc[...] / jnp.maximum(deg_ref[...], 1.0)


def kernel(source, query, fun, W0, b0, W1, b1, W2, b2):
    x = source[0]
    y = query[0]
    f_ch = fun.shape[1]
    pad_n = _NP - _N
    xp = jnp.concatenate(
        [x, jnp.full((pad_n, 3), 999.0, jnp.float32)], axis=0)
    yp = jnp.concatenate(
        [y, jnp.full((pad_n, 3), 999.0, jnp.float32)], axis=0)
    funp = jnp.concatenate(
        [fun, jnp.zeros((pad_n, f_ch), jnp.float32)], axis=0)
    xt8 = jnp.concatenate([xp.T, jnp.zeros((5, _NP), jnp.float32)], axis=0)
    w0y, w0x, w0f = W0[0:3], W0[3:6], W0[6:]

    nb1 = _NP // 1024
    full = lambda a: pl.BlockSpec(a.shape, lambda *_: (0,) * a.ndim)
    a_mat, c_mat = pl.pallas_call(
        _k1_body,
        grid=(nb1,),
        in_specs=[
            pl.BlockSpec((1024, 3), lambda i: (i, 0)),
            pl.BlockSpec((1024, f_ch), lambda i: (i, 0)),
            pl.BlockSpec((1024, 3), lambda i: (i, 0)),
            full(w0x), full(w0f), full(b0), full(w0y),
        ],
        out_specs=[
            pl.BlockSpec((1024, 256), lambda i: (i, 0)),
            pl.BlockSpec((1024, 256), lambda i: (i, 0)),
        ],
        out_shape=[
            jax.ShapeDtypeStruct((_NP, 256), jnp.float32),
            jax.ShapeDtypeStruct((_NP, 256), jnp.float32),
        ],
    )(xp, funp, yp, w0x, w0f, b0, w0y)

    nsb = 5
    sbw = _NP // nsb
    mask, deg2d = pl.pallas_call(
        functools.partial(_k2_body, nsb),
        grid=(_NW, nsb),
        in_specs=[
            pl.BlockSpec((_RPW, 3), lambda qb, sb: (qb, 0)),
            pl.BlockSpec((8, sbw), lambda qb, sb: (0, sb)),
        ],
        out_specs=[
            pl.BlockSpec((_RPW, sbw), lambda qb, sb: (qb, sb)),
            pl.BlockSpec((_RPW, 128), lambda qb, sb: (qb, 0)),
        ],
        out_shape=[
            jax.ShapeDtypeStruct((_NP, _NP), jnp.int32),
            jax.ShapeDtypeStruct((_NP, 128), jnp.float32),
        ],
        scratch_shapes=[pltpu.VMEM((_RPW, 128), jnp.float32)],
    )(yp, xt8)

    mesh = plsc.VectorSubcoreMesh(core_axis_name="c", subcore_axis_name="s")
    k3 = pl.kernel(
        _k3_body,
        mesh=mesh,
        out_type=[
            jax.ShapeDtypeStruct((_NW * _CAP, 256), jnp.float32),
            jax.ShapeDtypeStruct((_NW * _CAP, f_ch), jnp.float32),
            jax.ShapeDtypeStruct((_NW * _CAP,), jnp.int32),
        ],
        scratch_types=[
            pltpu.VMEM((8, 1280), jnp.int32),
            pltpu.VMEM((8, 1280), jnp.int32),
            pltpu.VMEM((_CAP,), jnp.int32),
            pltpu.VMEM((_CAP,), jnp.int32),
            pltpu.VMEM((_EB, 256), jnp.float32),
            pltpu.VMEM((_EB, f_ch), jnp.float32),
            pltpu.VMEM((_EB, 256), jnp.float32),
            pltpu.VMEM((_EB, f_ch), jnp.float32),
            pltpu.SemaphoreType.DMA,
            pltpu.SemaphoreType.DMA,
            pltpu.SemaphoreType.DMA,
            pltpu.SemaphoreType.DMA,
            pltpu.SemaphoreType.DMA,
            pltpu.SemaphoreType.DMA,
        ],
        compiler_params=pltpu.CompilerParams(needs_layout_passes=False),
    )
    hbuf, fbuf, edst = k3(mask, a_mat, funp)
    edst3 = edst.reshape(_NW * _NBLK4, 1, _EB4)

    outp = pl.pallas_call(
        _k4_body,
        grid=(_NW, _NBLK4),
        in_specs=[
            pl.BlockSpec((_EB4, 256), lambda w, b: (w * _NBLK4 + b, 0)),
            pl.BlockSpec((_EB4, f_ch), lambda w, b: (w * _NBLK4 + b, 0)),
            pl.BlockSpec((1, 1, _EB4), lambda w, b: (w * _NBLK4 + b, 0, 0)),
            pl.BlockSpec((_RPW, 256), lambda w, b: (w, 0)),
            pl.BlockSpec((_RPW, 128), lambda w, b: (w, 0)),
            full(W1), full(b1), full(W2), full(b2),
        ],
        out_specs=pl.BlockSpec((_RPW, f_ch), lambda w, b: (w, 0)),
        out_shape=jax.ShapeDtypeStruct((_NP, f_ch), jnp.float32),
        scratch_shapes=[pltpu.VMEM((_RPW, f_ch), jnp.float32)],
    )(hbuf, fbuf, edst3, c_mat, deg2d, W1, b1, W2, b2)
    return outp[:_N]
